# baseline (speedup 1.0000x reference)
import functools

import jax
import jax.numpy as jnp
from jax import lax
from jax.experimental import pallas as pl
from jax.experimental.pallas import tpu as pltpu
from jax.experimental.pallas import tpu_sc as plsc

NC = 2
NS = 16
NW = NC * NS
CHUNK = 128


def _feature_transform(x, weight):
    n, f = x.shape
    d = weight.shape[0]
    bm = 1000 if n % 1000 == 0 else 8
    grid = (n // bm,)

    def body(x_ref, w_ref, o_ref):
        o_ref[...] = lax.dot_general(
            x_ref[...], w_ref[...], (((1,), (1,)), ((), ())),
            preferred_element_type=jnp.float32)

    return pl.pallas_call(
        body,
        grid=grid,
        in_specs=[
            pl.BlockSpec((bm, f), lambda i: (i, 0)),
            pl.BlockSpec((d, f), lambda i: (0, 0)),
        ],
        out_specs=pl.BlockSpec((bm, d), lambda i: (i, 0)),
        out_shape=jax.ShapeDtypeStruct((n, d), jnp.float32),
    )(x, weight)


GCHUNKS = 16


def _sc_aggregate(h, src3, dst3, rows, cpt):
    n, d = h.shape
    rows_per_sub = rows // NS
    zblocks = rows_per_sub // CHUNK
    groups = cpt // GCHUNKS
    mesh = plsc.VectorSubcoreMesh(core_axis_name="c", subcore_axis_name="s")

    @functools.partial(
        pl.kernel,
        out_type=jax.ShapeDtypeStruct((NC, rows, d), jnp.float32),
        mesh=mesh,
        scratch_types=[
            pltpu.VMEM((GCHUNKS, CHUNK), jnp.int32),
            pltpu.VMEM((GCHUNKS, CHUNK), jnp.int32),
            pltpu.VMEM((CHUNK, d), jnp.float32),
            pltpu.VMEM((CHUNK, d), jnp.float32),
            pltpu.VMEM_SHARED((rows, d), jnp.float32),
            pltpu.SemaphoreType.DMA,
            pltpu.SemaphoreType.DMA,
        ],
    )
    def agg(h_hbm, src_hbm, dst_hbm, out_hbm, idx_s, idx_d, buf0, buf1,
            part, sem0, sem1):
        cid = lax.axis_index("c")
        sid = lax.axis_index("s")
        wid = cid * NS + sid

        def zrow(r, _):
            def zlane(i, _):
                buf0[r, pl.ds(i * 16, 16)] = jnp.zeros((16,), jnp.float32)
                return 0
            return lax.fori_loop(0, d // 16, zlane, 0)
        lax.fori_loop(0, CHUNK, zrow, 0)

        def zblock(b, _):
            pltpu.sync_copy(
                buf0, part.at[pl.ds(sid * rows_per_sub + b * CHUNK, CHUNK)])
            return 0
        lax.fori_loop(0, zblocks, zblock, 0)
        plsc.subcore_barrier()

        def group(g, _):
            pltpu.sync_copy(src_hbm.at[wid, pl.ds(g * GCHUNKS, GCHUNKS)],
                            idx_s)
            pltpu.sync_copy(dst_hbm.at[wid, pl.ds(g * GCHUNKS, GCHUNKS)],
                            idx_d)
            def start(j, buf, sem):
                pltpu.async_copy(
                    h_hbm.at[idx_s.at[j, pl.ds(0, 64)]],
                    buf.at[pl.ds(0, 64)], sem)
                pltpu.async_copy(
                    h_hbm.at[idx_s.at[j, pl.ds(64, 64)]],
                    buf.at[pl.ds(64, 64)], sem)

            def wait(j, buf, sem):
                pltpu.make_async_copy(h_hbm.at[idx_s.at[j]], buf, sem).wait()

            start(0, buf0, sem0)

            def body(i, _):
                j = 2 * i
                start(j + 1, buf1, sem1)
                wait(j, buf0, sem0)
                pltpu.sync_copy(buf0, part.at[idx_d.at[j]], add=True)

                @pl.when(j + 2 < GCHUNKS)
                def _():
                    start(j + 2, buf0, sem0)
                wait(j + 1, buf1, sem1)
                pltpu.sync_copy(buf1, part.at[idx_d.at[j + 1]], add=True)
                return 0
            return lax.fori_loop(0, GCHUNKS // 2, body, 0)
        lax.fori_loop(0, groups, group, 0)
        plsc.subcore_barrier()

        pltpu.sync_copy(
            part.at[pl.ds(sid * rows_per_sub, rows_per_sub)],
            out_hbm.at[cid, pl.ds(sid * rows_per_sub, rows_per_sub)])

    return agg(h, src3, dst3)


def _sum_partials(partials, n):
    _, rows, d = partials.shape
    bm = 1000 if n % 1000 == 0 else 8
    grid = (n // bm,)

    def body(p0_ref, p1_ref, o_ref):
        o_ref[...] = p0_ref[0] + p1_ref[0]

    return pl.pallas_call(
        body,
        grid=grid,
        in_specs=[
            pl.BlockSpec((1, bm, d), lambda i: (0, i, 0)),
            pl.BlockSpec((1, bm, d), lambda i: (1, i, 0)),
        ],
        out_specs=pl.BlockSpec((bm, d), lambda i: (i, 0)),
        out_shape=jax.ShapeDtypeStruct((n, d), jnp.float32),
    )(partials, partials)


def kernel(x, edge_index, weight):
    n, _ = x.shape
    e = edge_index.shape[1]
    d = weight.shape[0]

    h = _feature_transform(x, weight)

    cpt = -(-e // (NW * CHUNK))
    cpt = -(-cpt // GCHUNKS) * GCHUNKS
    e_pad = NW * cpt * CHUNK
    rows = -(-(n + 1) // (NS * CHUNK)) * NS * CHUNK
    src = edge_index[0]
    dst = edge_index[1]
    if e_pad > e:
        pad_dst = n + (jnp.arange(e_pad - e, dtype=jnp.int32) % (rows - n))
        src = jnp.concatenate([src, jnp.zeros((e_pad - e,), jnp.int32)])
        dst = jnp.concatenate([dst, pad_dst])
    src3 = src.reshape(NW, cpt, CHUNK)
    dst3 = dst.reshape(NW, cpt, CHUNK)
    partials = _sc_aggregate(h, src3, dst3, rows, cpt)
    return _sum_partials(partials, n)


# speedup vs baseline: 1.0723x; 1.0723x over previous
import functools

import jax
import jax.numpy as jnp
from jax import lax
from jax.experimental import pallas as pl
from jax.experimental.pallas import tpu as pltpu
from jax.experimental.pallas import tpu_sc as plsc

NC = 2
NS = 16
NW = NC * NS
CHUNK = 128


def _feature_transform(x, weight):
    n, f = x.shape
    d = weight.shape[0]
    bm = 1000 if n % 1000 == 0 else 8
    grid = (n // bm,)

    def body(x_ref, w_ref, o_ref):
        o_ref[...] = lax.dot_general(
            x_ref[...], w_ref[...], (((1,), (1,)), ((), ())),
            preferred_element_type=jnp.float32)

    return pl.pallas_call(
        body,
        grid=grid,
        in_specs=[
            pl.BlockSpec((bm, f), lambda i: (i, 0)),
            pl.BlockSpec((d, f), lambda i: (0, 0)),
        ],
        out_specs=pl.BlockSpec((bm, d), lambda i: (i, 0)),
        out_shape=jax.ShapeDtypeStruct((n, d), jnp.float32),
    )(x, weight)


GCHUNKS = 16


def _sc_aggregate(h, src3, dst3, rows, g0, g1):
    n, d = h.shape
    rows_per_sub = rows // NS
    zblocks = rows_per_sub // CHUNK
    mesh = plsc.VectorSubcoreMesh(core_axis_name="c", subcore_axis_name="s")

    @functools.partial(
        pl.kernel,
        out_type=jax.ShapeDtypeStruct((NC, rows, d), jnp.float32),
        mesh=mesh,
        scratch_types=[
            pltpu.VMEM((GCHUNKS, CHUNK), jnp.int32),
            pltpu.VMEM((GCHUNKS, CHUNK), jnp.int32),
            pltpu.VMEM((CHUNK, d), jnp.float32),
            pltpu.VMEM((CHUNK, d), jnp.float32),
            pltpu.VMEM_SHARED((rows, d), jnp.float32),
            pltpu.SemaphoreType.DMA,
            pltpu.SemaphoreType.DMA,
        ],
    )
    def agg(h_hbm, src_hbm, dst_hbm, out_hbm, idx_s, idx_d, buf0, buf1,
            part, sem0, sem1):
        cid = lax.axis_index("c")
        sid = lax.axis_index("s")
        wid = cid * NS + sid

        def zrow(r, _):
            def zlane(i, _):
                buf0[r, pl.ds(i * 16, 16)] = jnp.zeros((16,), jnp.float32)
                return 0
            return lax.fori_loop(0, d // 16, zlane, 0)
        lax.fori_loop(0, CHUNK, zrow, 0)

        def zblock(b, _):
            pltpu.sync_copy(
                buf0, part.at[pl.ds(sid * rows_per_sub + b * CHUNK, CHUNK)])
            return 0
        lax.fori_loop(0, zblocks, zblock, 0)
        plsc.subcore_barrier()

        def group(g, _):
            pltpu.sync_copy(src_hbm.at[wid, pl.ds(g * GCHUNKS, GCHUNKS)],
                            idx_s)
            pltpu.sync_copy(dst_hbm.at[wid, pl.ds(g * GCHUNKS, GCHUNKS)],
                            idx_d)
            def start(j, buf, sem):
                pltpu.async_copy(
                    h_hbm.at[idx_s.at[j, pl.ds(0, 64)]],
                    buf.at[pl.ds(0, 64)], sem)
                pltpu.async_copy(
                    h_hbm.at[idx_s.at[j, pl.ds(64, 64)]],
                    buf.at[pl.ds(64, 64)], sem)

            def wait(j, buf, sem):
                pltpu.make_async_copy(h_hbm.at[idx_s.at[j]], buf, sem).wait()

            start(0, buf0, sem0)

            def body(i, _):
                j = 2 * i
                start(j + 1, buf1, sem1)
                wait(j, buf0, sem0)
                pltpu.sync_copy(buf0, part.at[idx_d.at[j]], add=True)

                @pl.when(j + 2 < GCHUNKS)
                def _():
                    start(j + 2, buf0, sem0)
                wait(j + 1, buf1, sem1)
                pltpu.sync_copy(buf1, part.at[idx_d.at[j + 1]], add=True)
                return 0
            return lax.fori_loop(0, GCHUNKS // 2, body, 0)
        ng = jnp.where(cid == 0, g0, g1)
        lax.fori_loop(0, ng, group, 0)
        plsc.subcore_barrier()

        pltpu.sync_copy(
            part.at[pl.ds(sid * rows_per_sub, rows_per_sub)],
            out_hbm.at[cid, pl.ds(sid * rows_per_sub, rows_per_sub)])

    return agg(h, src3, dst3)


def _sum_partials(partials, n):
    _, rows, d = partials.shape
    bm = 1000 if n % 1000 == 0 else 8
    grid = (n // bm,)

    def body(p0_ref, p1_ref, o_ref):
        o_ref[...] = p0_ref[0] + p1_ref[0]

    return pl.pallas_call(
        body,
        grid=grid,
        in_specs=[
            pl.BlockSpec((1, bm, d), lambda i: (0, i, 0)),
            pl.BlockSpec((1, bm, d), lambda i: (1, i, 0)),
        ],
        out_specs=pl.BlockSpec((bm, d), lambda i: (i, 0)),
        out_shape=jax.ShapeDtypeStruct((n, d), jnp.float32),
    )(partials, partials)


def kernel(x, edge_index, weight):
    n, _ = x.shape
    e = edge_index.shape[1]
    d = weight.shape[0]

    h = _feature_transform(x, weight)

    gedges = NS * GCHUNKS * CHUNK
    gtot = -(-e // gedges)
    g0 = max(1, min(gtot - 1, round(gtot * 0.8)))
    g1 = gtot - g0
    e0 = g0 * gedges
    cpt_max = max(g0, g1) * GCHUNKS
    e_pad = gtot * gedges
    rows = -(-(n + 1) // (NS * CHUNK)) * NS * CHUNK
    src = edge_index[0]
    dst = edge_index[1]
    if e_pad > e:
        pad_dst = n + (jnp.arange(e_pad - e, dtype=jnp.int32) % (rows - n))
        src = jnp.concatenate([src, jnp.zeros((e_pad - e,), jnp.int32)])
        dst = jnp.concatenate([dst, pad_dst])

    def to3d(a):
        p0 = a[:e0].reshape(NS, g0 * GCHUNKS, CHUNK)
        p1 = a[e0:].reshape(NS, g1 * GCHUNKS, CHUNK)
        p0 = jnp.pad(p0, ((0, 0), (0, cpt_max - g0 * GCHUNKS), (0, 0)))
        p1 = jnp.pad(p1, ((0, 0), (0, cpt_max - g1 * GCHUNKS), (0, 0)))
        return jnp.concatenate([p0, p1], axis=0)

    partials = _sc_aggregate(h, to3d(src), to3d(dst), rows, g0, g1)
    return _sum_partials(partials, n)
